# baseline (speedup 1.0000x reference)
import jax
import jax.numpy as jnp
from jax.experimental import pallas as pl

_B = 1024
_PH, _PW = 32, 16
_F = _PH * _PW
_ROWS_PER_BLOCK = 256


def _zero_fill_kernel(pos_ref, neg_ref):
    pos_ref[...] = jnp.zeros_like(pos_ref)
    neg_ref[...] = jnp.zeros_like(neg_ref)


def kernel(weight, encoder_out, split_thresh):
    del weight, encoder_out, split_thresh
    b = _B
    grid = (b // _ROWS_PER_BLOCK,)
    spec = pl.BlockSpec((_ROWS_PER_BLOCK, _F), lambda i: (i, 0))
    pos, neg = pl.pallas_call(
        _zero_fill_kernel,
        grid=grid,
        out_specs=(spec, spec),
        out_shape=(
            jax.ShapeDtypeStruct((b, _F), jnp.float32),
            jax.ShapeDtypeStruct((b, _F), jnp.float32),
        ),
    )()
    return (pos.reshape(b, _PH, _PW), neg.reshape(b, _PH, _PW))
